# baseline (speedup 1.0000x reference)
import functools

import jax
import jax.numpy as jnp
from jax import lax
from jax.experimental import pallas as pl
from jax.experimental.pallas import tpu as pltpu
from jax.experimental.pallas import tpu_sc as plsc

NUM_PARTITIONS = 8
P = NUM_PARTITIONS
NW = 32
CX = 32
NCHK = 4
ZR = 32
NZPT = P - 1


def _sc_body(Tw, K, D, x_hbm, rows_hbm, zrows_hbm, out_hbm,
             xbuf, idxv, zidx, zbuf,
             zs0, zs1, zs2, zs3, xs0, xs1, ss0, ss1):
    zsems = [zs0, zs1, zs2, zs3]
    xsems = [xs0, xs1]
    ssems = [ss0, ss1]
    wid = lax.axis_index("s") * 2 + lax.axis_index("c")
    tbase = wid * Tw
    NV = D // 16

    def zrow(r, carry):
        for v in range(NV):
            zbuf[r, pl.ds(v * 16, 16)] = jnp.zeros((16,), jnp.float32)
        return carry
    lax.fori_loop(0, ZR, zrow, 0)

    for k in range(K):
        pltpu.sync_copy(rows_hbm.at[k, wid], idxv.at[k])
    for j in range(NZPT):
        pltpu.sync_copy(zrows_hbm.at[j, wid], zidx.at[j])

    def zero_chunk(ci, do_start):
        for j in range(NZPT):
            cp = pltpu.make_async_copy(
                zbuf, out_hbm.at[zidx.at[j, ci]], zsems[ci])
            if do_start:
                cp.start()
            else:
                cp.wait()

    def copy_x(ci, do_start):
        cp = pltpu.make_async_copy(
            x_hbm.at[pl.ds(tbase + ci * CX, CX)],
            xbuf.at[ci % 2], xsems[ci % 2])
        if do_start:
            cp.start()
        else:
            cp.wait()

    def scatter(ci, do_start):
        for k in range(K):
            cp = pltpu.make_async_copy(
                xbuf.at[ci % 2], out_hbm.at[idxv.at[k, ci]], ssems[ci % 2])
            if do_start:
                cp.start()
            else:
                cp.wait()

    for ci in range(NCHK):
        zero_chunk(ci, True)
    copy_x(0, True)
    for ci in range(NCHK):
        if ci + 1 < NCHK:
            if ci >= 1:
                scatter(ci - 1, False)
            copy_x(ci + 1, True)
        zero_chunk(ci, False)
        copy_x(ci, False)
        scatter(ci, True)
    scatter(NCHK - 2, False)
    scatter(NCHK - 1, False)


def kernel(x, partition_indices):
    B, S, D = x.shape
    T = B * S
    K = partition_indices.shape[-1]
    Tw = T // NW
    x2d = x.reshape(T, D)
    idx = partition_indices.reshape(T, K).astype(jnp.int32)
    tok = jnp.arange(T, dtype=jnp.int32)[:, None]
    rows = (tok * P + idx).T.reshape(K, NW, NCHK, CX)
    mask = jnp.zeros((T, P), jnp.int32).at[tok, idx].set(1)
    zp = jnp.argsort(mask, axis=1)[:, :NZPT].astype(jnp.int32)
    zrows = (tok * P + zp).T.reshape(NZPT, NW, NCHK, CX)

    body = functools.partial(_sc_body, Tw, K, D)
    out = pl.kernel(
        body,
        out_type=jax.ShapeDtypeStruct((T * P, D), jnp.float32),
        mesh=plsc.VectorSubcoreMesh(core_axis_name="c", subcore_axis_name="s"),
        scratch_types=[
            pltpu.VMEM((2, CX, D), jnp.float32),
            pltpu.VMEM((K, NCHK, CX), jnp.int32),
            pltpu.VMEM((NZPT, NCHK, CX), jnp.int32),
            pltpu.VMEM((ZR, D), jnp.float32),
            pltpu.SemaphoreType.DMA,
            pltpu.SemaphoreType.DMA,
            pltpu.SemaphoreType.DMA,
            pltpu.SemaphoreType.DMA,
            pltpu.SemaphoreType.DMA,
            pltpu.SemaphoreType.DMA,
            pltpu.SemaphoreType.DMA,
            pltpu.SemaphoreType.DMA,
        ],
    )(x2d, rows, zrows)
    return out.reshape(B, S, P, D)


# speedup vs baseline: 1.4487x; 1.4487x over previous
import functools

import jax
import jax.numpy as jnp
from jax import lax
from jax.experimental import pallas as pl
from jax.experimental.pallas import tpu as pltpu
from jax.experimental.pallas import tpu_sc as plsc

NUM_PARTITIONS = 8
P = NUM_PARTITIONS
NW = 32
CX = 32
NCHK = 4
ZR = 32


def _sc_body(Tw, K, D, x_hbm, rows_hbm, out_hbm,
             xbuf, idxv, zbuf, zs0, zs1, zs2, zs3, xs0, xs1, ss0, ss1):
    zsems = [zs0, zs1, zs2, zs3]
    xsems = [xs0, xs1]
    ssems = [ss0, ss1]
    wid = lax.axis_index("s") * 2 + lax.axis_index("c")
    tbase = wid * Tw
    NV = D // 16
    NZ = (CX * P) // ZR

    def zrow(r, carry):
        for v in range(NV):
            zbuf[r, pl.ds(v * 16, 16)] = jnp.zeros((16,), jnp.float32)
        return carry
    lax.fori_loop(0, ZR, zrow, 0)

    for k in range(K):
        pltpu.sync_copy(rows_hbm.at[k, wid], idxv.at[k])

    def zero_chunk(ci, do_start):
        base = (tbase + ci * CX) * P
        for j in range(NZ):
            cp = pltpu.make_async_copy(
                zbuf, out_hbm.at[pl.ds(base + j * ZR, ZR)], zsems[ci])
            if do_start:
                cp.start()
            else:
                cp.wait()

    def copy_x(ci, do_start):
        cp = pltpu.make_async_copy(
            x_hbm.at[pl.ds(tbase + ci * CX, CX)],
            xbuf.at[ci % 2], xsems[ci % 2])
        if do_start:
            cp.start()
        else:
            cp.wait()

    def scatter(ci, do_start):
        for k in range(K):
            cp = pltpu.make_async_copy(
                xbuf.at[ci % 2], out_hbm.at[idxv.at[k, ci]], ssems[ci % 2])
            if do_start:
                cp.start()
            else:
                cp.wait()

    zero_chunk(0, True)
    copy_x(0, True)
    for ci in range(NCHK):
        if ci + 1 < NCHK:
            zero_chunk(ci + 1, True)
            if ci >= 1:
                scatter(ci - 1, False)
            copy_x(ci + 1, True)
        zero_chunk(ci, False)
        copy_x(ci, False)
        scatter(ci, True)
    scatter(NCHK - 2, False)
    scatter(NCHK - 1, False)


def kernel(x, partition_indices):
    B, S, D = x.shape
    T = B * S
    K = partition_indices.shape[-1]
    Tw = T // NW
    x2d = x.reshape(T, D)
    idx = partition_indices.reshape(T, K).astype(jnp.int32)
    tok = jnp.arange(T, dtype=jnp.int32)[:, None]
    rows = (tok * P + idx).T.reshape(K, NW, NCHK, CX)

    body = functools.partial(_sc_body, Tw, K, D)
    out = pl.kernel(
        body,
        out_type=jax.ShapeDtypeStruct((T * P, D), jnp.float32),
        mesh=plsc.VectorSubcoreMesh(core_axis_name="c", subcore_axis_name="s"),
        scratch_types=[
            pltpu.VMEM((2, CX, D), jnp.float32),
            pltpu.VMEM((K, NCHK, CX), jnp.int32),
            pltpu.VMEM((ZR, D), jnp.float32),
            pltpu.SemaphoreType.DMA,
            pltpu.SemaphoreType.DMA,
            pltpu.SemaphoreType.DMA,
            pltpu.SemaphoreType.DMA,
            pltpu.SemaphoreType.DMA,
            pltpu.SemaphoreType.DMA,
            pltpu.SemaphoreType.DMA,
            pltpu.SemaphoreType.DMA,
        ],
    )(x2d, rows)
    return out.reshape(B, S, P, D)
